# baseline (speedup 1.0000x reference)
import functools

import jax
import jax.numpy as jnp
from jax import lax
from jax.experimental import pallas as pl
from jax.experimental.pallas import tpu as pltpu
from jax.experimental.pallas import tpu_sc as plsc

_NC = 2
_NS = 16
_DS = 4
_KE = 8000


@functools.cache
def _make_agg(N, NP, D, E, with_deg):
    NW = _NC * _NS
    assert D == _DS * NW and E % _KE == 0 and _KE % 16 == 0
    assert NP % 128 == 0 and N <= NP
    CH = E // _KE
    SLAB = _DS * NP
    ACC = (_DS + (1 if with_deg else 0)) * NP

    mesh = plsc.VectorSubcoreMesh(
        core_axis_name="c", subcore_axis_name="s",
        num_cores=_NC, num_subcores=_NS)

    out_type = [jax.ShapeDtypeStruct((NW * SLAB,), jnp.float32)]
    if with_deg:
        out_type.append(jax.ShapeDtypeStruct((NP,), jnp.float32))
    out_type = tuple(out_type) if with_deg else out_type[0]
    NACC = _DS + (1 if with_deg else 0)
    scratch = [
        pltpu.VMEM((_KE,), jnp.int32),
        pltpu.VMEM((_KE,), jnp.int32),
        pltpu.VMEM((_KE,), jnp.int32),
        pltpu.VMEM((_KE,), jnp.int32),
        pltpu.SemaphoreType.DMA,
        pltpu.SemaphoreType.DMA,
    ]

    def body(xt_h, src_h, dst_h, out_h, *rest):
        if with_deg:
            deg_h, sidx, didx, sidx2, didx2, sem, sem2 = rest
        else:
            sidx, didx, sidx2, didx2, sem, sem2 = rest
        cid = lax.axis_index("c")
        sid = lax.axis_index("s")
        wg = cid * _NS + sid
        sbase = pl.multiple_of(wg * SLAB, SLAB)

        def scoped(*bufs_np):
            slabs = bufs_np[:_DS]
            accs = bufs_np[_DS:]

            for c in range(_DS):
                base = pl.multiple_of(wg * SLAB + c * NP, NP)
                pltpu.sync_copy(xt_h.at[pl.ds(base, NP)], slabs[c])

            def zrow(i, carry):
                for a in accs:
                    a[pl.ds(i * 16, 16)] = jnp.zeros((16,), jnp.float32)
                return carry
            lax.fori_loop(0, NP // 16, zrow, 0)

            ones = jnp.ones((16,), jnp.float32)
            bufs = ((sidx, didx, sem), (sidx2, didx2, sem2))

            def start_fetch(c, buf):
                si, di, sm = buf
                rel = jnp.minimum(c * _KE, E - _KE)
                base = pl.multiple_of(rel, _KE)
                pltpu.async_copy(src_h.at[pl.ds(base, _KE)], si, sm)
                pltpu.async_copy(dst_h.at[pl.ds(base, _KE)], di, sm)

            def wait_fetch(buf):
                si, di, sm = buf
                pltpu.make_async_copy(src_h.at[pl.ds(0, _KE)], si, sm).wait()
                pltpu.make_async_copy(dst_h.at[pl.ds(0, _KE)], di, sm).wait()

            def run_chunk(buf):
                si, di, sm = buf

                @plsc.parallel_loop(0, _KE // 16, unroll=4)
                def group(g):
                    s16 = si[pl.ds(g * 16, 16)]
                    d16 = di[pl.ds(g * 16, 16)]
                    for c in range(_DS):
                        v = plsc.load_gather(slabs[c], [s16])
                        plsc.addupdate_scatter(accs[c], [d16], v)
                    if with_deg:
                        plsc.addupdate_scatter(accs[_DS], [d16], ones)

            start_fetch(0, bufs[0])
            start_fetch(1, bufs[1])

            def chunkpair(p, carry):
                c = p * 2
                wait_fetch(bufs[0])
                run_chunk(bufs[0])
                start_fetch(c + 2, bufs[0])
                wait_fetch(bufs[1])
                run_chunk(bufs[1])
                start_fetch(c + 3, bufs[1])
                return carry

            assert CH % 2 == 0
            lax.fori_loop(0, CH // 2, chunkpair, 0)
            wait_fetch(bufs[0])
            wait_fetch(bufs[1])

            for c in range(_DS):
                base = pl.multiple_of(wg * SLAB + c * NP, NP)
                pltpu.sync_copy(accs[c], out_h.at[pl.ds(base, NP)])

            if with_deg:
                @pl.when(wg == 0)
                def _():
                    pltpu.sync_copy(accs[_DS], deg_h)

        pl.run_scoped(
            scoped,
            *([pltpu.VMEM((NP,), jnp.float32)] * (_DS + NACC)))

    return pl.kernel(
        body, out_type=out_type, mesh=mesh, scratch_types=scratch,
        compiler_params=pltpu.CompilerParams(needs_layout_passes=False))


@functools.cache
def _make_xt(N, NP, D):
    def body(x_ref, o_ref):
        xt = x_ref[:].T
        o_ref[:] = jnp.concatenate(
            [xt, jnp.zeros((D, NP - N), jnp.float32)], axis=1)

    return pl.pallas_call(
        body, out_shape=jax.ShapeDtypeStruct((D, NP), jnp.float32))


@functools.cache
def _make_layer(N, NP, D, H):
    def body(at_ref, d_ref, x_ref, wl_ref, wr_ref, bl_ref, o_ref, ot_ref):
        inv = 1.0 / jnp.maximum(d_ref[:, :N], 1.0)
        a = at_ref[:, :N] * inv
        dn = (((0,), (0,)), ((), ()))
        h = lax.dot_general(a, wl_ref[:], dn,
                            preferred_element_type=jnp.float32)
        h = h + jnp.dot(x_ref[:], wr_ref[:],
                        preferred_element_type=jnp.float32)
        h = jnp.maximum(h + bl_ref[:], 0.0)
        o_ref[:] = h
        ot_ref[:] = jnp.concatenate(
            [h.T, jnp.zeros((H, NP - N), jnp.float32)], axis=1)

    return pl.pallas_call(
        body, out_shape=(jax.ShapeDtypeStruct((N, H), jnp.float32),
                         jax.ShapeDtypeStruct((H, NP), jnp.float32)))


@functools.cache
def _make_final(N, NP, H, G, C):
    def body(at_ref, d_ref, x_ref, wl_ref, wr_ref, bl_ref,
             b_ref, wc_ref, bc_ref, o_ref):
        inv = 1.0 / jnp.maximum(d_ref[:, :N], 1.0)
        a = at_ref[:, :N] * inv
        dn = (((0,), (0,)), ((), ()))
        h = lax.dot_general(a, wl_ref[:], dn,
                            preferred_element_type=jnp.float32)
        h = h + jnp.dot(x_ref[:], wr_ref[:],
                        preferred_element_type=jnp.float32)
        h = jnp.maximum(h + bl_ref[:], 0.0)

        onehot = (b_ref[:] == lax.broadcasted_iota(jnp.int32, (N, G), 1))
        onehot = onehot.astype(jnp.float32)
        sums = lax.dot_general(onehot, h, dn,
                               preferred_element_type=jnp.float32)
        cnts = lax.dot_general(onehot, jnp.ones((N, 1), jnp.float32), dn,
                               preferred_element_type=jnp.float32)
        pooled = sums / jnp.maximum(cnts, 1.0)
        o_ref[:] = (jnp.dot(pooled, wc_ref[:],
                            preferred_element_type=jnp.float32) + bc_ref[:])

    return pl.pallas_call(
        body, out_shape=jax.ShapeDtypeStruct((G, C), jnp.float32))


def kernel(x, edge_index, batch, Wl1, bl1, Wr1, Wl2, bl2, Wr2, Wc, bc):
    N, D = x.shape
    H = Wl1.shape[1]
    C = Wc.shape[1]
    G = 128
    E = edge_index.shape[1]
    src = edge_index[0]
    dst = edge_index[1]
    NP = ((N + 127) // 128) * 128

    xt = _make_xt(N, NP, D)(x).reshape(-1)
    aggt1, deg = _make_agg(N, NP, D, E, True)(xt, src, dst)
    h1, h1t = _make_layer(N, NP, D, H)(
        aggt1.reshape(D, NP), deg.reshape(1, NP), x, Wl1, Wr1,
        bl1.reshape(1, H))
    aggt2 = _make_agg(N, NP, H, E, False)(h1t.reshape(-1), src, dst)
    out = _make_final(N, NP, H, G, C)(
        aggt2.reshape(H, NP), deg.reshape(1, NP), h1, Wl2, Wr2,
        bl2.reshape(1, H), batch.reshape(N, 1), Wc, bc.reshape(1, C))
    return out


# speedup vs baseline: 1.0297x; 1.0297x over previous
import functools

import jax
import jax.numpy as jnp
from jax import lax
from jax.experimental import pallas as pl
from jax.experimental.pallas import tpu as pltpu
from jax.experimental.pallas import tpu_sc as plsc

_NC = 2
_NS = 16
_DS = 4
_KE = 8000


@functools.cache
def _make_agg(N, NP, D, E, with_deg):
    NW = _NC * _NS
    assert D == _DS * NW and E % _KE == 0 and _KE % 16 == 0
    assert NP % 128 == 0 and N <= NP
    CH = E // _KE
    SLAB = _DS * NP
    ACC = (_DS + (1 if with_deg else 0)) * NP

    mesh = plsc.VectorSubcoreMesh(
        core_axis_name="c", subcore_axis_name="s",
        num_cores=_NC, num_subcores=_NS)

    out_type = [jax.ShapeDtypeStruct((NW * SLAB,), jnp.float32)]
    if with_deg:
        out_type.append(jax.ShapeDtypeStruct((NP,), jnp.float32))
    out_type = tuple(out_type) if with_deg else out_type[0]
    scratch = [
        pltpu.VMEM((_KE,), jnp.int32),
        pltpu.VMEM((_KE,), jnp.int32),
        pltpu.VMEM((_KE,), jnp.int32),
        pltpu.VMEM((_KE,), jnp.int32),
        pltpu.VMEM((SLAB,), jnp.float32),
        pltpu.VMEM((ACC,), jnp.float32),
        pltpu.SemaphoreType.DMA,
        pltpu.SemaphoreType.DMA,
    ]

    def body(xt_h, src_h, dst_h, out_h, *rest):
        if with_deg:
            (deg_h, sidx, didx, sidx2, didx2, slab, acc,
             sem, sem2) = rest
        else:
            sidx, didx, sidx2, didx2, slab, acc, sem, sem2 = rest
        cid = lax.axis_index("c")
        sid = lax.axis_index("s")
        wg = cid * _NS + sid

        sbase = pl.multiple_of(wg * SLAB, SLAB)
        pltpu.sync_copy(xt_h.at[pl.ds(sbase, SLAB)], slab)

        def zrow(i, carry):
            acc[pl.ds(i * 16, 16)] = jnp.zeros((16,), jnp.float32)
            return carry
        lax.fori_loop(0, ACC // 16, zrow, 0)

        ones = jnp.ones((16,), jnp.float32)
        bufs = ((sidx, didx, sem), (sidx2, didx2, sem2))

        def start_fetch(c, buf):
            si, di, sm = buf
            rel = jnp.minimum(c * _KE, E - _KE)
            base = pl.multiple_of(rel, _KE)
            pltpu.async_copy(src_h.at[pl.ds(base, _KE)], si, sm)
            pltpu.async_copy(dst_h.at[pl.ds(base, _KE)], di, sm)

        def wait_fetch(buf):
            si, di, sm = buf
            pltpu.make_async_copy(src_h.at[pl.ds(0, _KE)], si, sm).wait()
            pltpu.make_async_copy(dst_h.at[pl.ds(0, _KE)], di, sm).wait()

        def run_chunk(buf):
            si, di, sm = buf

            @plsc.parallel_loop(0, _KE // 16, unroll=4)
            def group(g):
                s16 = si[pl.ds(g * 16, 16)]
                d16 = di[pl.ds(g * 16, 16)]
                for c in range(_DS):
                    v = plsc.load_gather(slab, [s16 + (c * NP)])
                    plsc.addupdate_scatter(acc, [d16 + (c * NP)], v)
                if with_deg:
                    plsc.addupdate_scatter(acc, [d16 + (_DS * NP)], ones)

        start_fetch(0, bufs[0])
        start_fetch(1, bufs[1])

        def chunkpair(p, carry):
            c = p * 2
            wait_fetch(bufs[0])
            run_chunk(bufs[0])
            start_fetch(c + 2, bufs[0])
            wait_fetch(bufs[1])
            run_chunk(bufs[1])
            start_fetch(c + 3, bufs[1])
            return carry

        assert CH % 2 == 0
        lax.fori_loop(0, CH // 2, chunkpair, 0)
        wait_fetch(bufs[0])
        wait_fetch(bufs[1])

        pltpu.sync_copy(acc.at[pl.ds(0, SLAB)], out_h.at[pl.ds(sbase, SLAB)])

        if with_deg:
            @pl.when(wg == 0)
            def _():
                pltpu.sync_copy(acc.at[pl.ds(_DS * NP, NP)], deg_h)

    return pl.kernel(
        body, out_type=out_type, mesh=mesh, scratch_types=scratch,
        compiler_params=pltpu.CompilerParams(needs_layout_passes=False))


@functools.cache
def _make_xt(N, NP, D):
    def body(x_ref, o_ref):
        xt = x_ref[:].T
        o_ref[:] = jnp.concatenate(
            [xt, jnp.zeros((D, NP - N), jnp.float32)], axis=1)

    return pl.pallas_call(
        body, out_shape=jax.ShapeDtypeStruct((D, NP), jnp.float32))


@functools.cache
def _make_layer(N, NP, D, H):
    def body(at_ref, d_ref, x_ref, wl_ref, wr_ref, bl_ref, o_ref, ot_ref):
        inv = 1.0 / jnp.maximum(d_ref[:, :N], 1.0)
        a = at_ref[:, :N] * inv
        dn = (((0,), (0,)), ((), ()))
        h = lax.dot_general(a, wl_ref[:], dn,
                            preferred_element_type=jnp.float32)
        h = h + jnp.dot(x_ref[:], wr_ref[:],
                        preferred_element_type=jnp.float32)
        h = jnp.maximum(h + bl_ref[:], 0.0)
        o_ref[:] = h
        ot_ref[:] = jnp.concatenate(
            [h.T, jnp.zeros((H, NP - N), jnp.float32)], axis=1)

    return pl.pallas_call(
        body, out_shape=(jax.ShapeDtypeStruct((N, H), jnp.float32),
                         jax.ShapeDtypeStruct((H, NP), jnp.float32)))


@functools.cache
def _make_final(N, NP, H, G, C):
    def body(at_ref, d_ref, x_ref, wl_ref, wr_ref, bl_ref,
             b_ref, wc_ref, bc_ref, o_ref):
        inv = 1.0 / jnp.maximum(d_ref[:, :N], 1.0)
        a = at_ref[:, :N] * inv
        dn = (((0,), (0,)), ((), ()))
        h = lax.dot_general(a, wl_ref[:], dn,
                            preferred_element_type=jnp.float32)
        h = h + jnp.dot(x_ref[:], wr_ref[:],
                        preferred_element_type=jnp.float32)
        h = jnp.maximum(h + bl_ref[:], 0.0)

        onehot = (b_ref[:] == lax.broadcasted_iota(jnp.int32, (N, G), 1))
        onehot = onehot.astype(jnp.float32)
        sums = lax.dot_general(onehot, h, dn,
                               preferred_element_type=jnp.float32)
        cnts = lax.dot_general(onehot, jnp.ones((N, 1), jnp.float32), dn,
                               preferred_element_type=jnp.float32)
        pooled = sums / jnp.maximum(cnts, 1.0)
        o_ref[:] = (jnp.dot(pooled, wc_ref[:],
                            preferred_element_type=jnp.float32) + bc_ref[:])

    return pl.pallas_call(
        body, out_shape=jax.ShapeDtypeStruct((G, C), jnp.float32))


def kernel(x, edge_index, batch, Wl1, bl1, Wr1, Wl2, bl2, Wr2, Wc, bc):
    N, D = x.shape
    H = Wl1.shape[1]
    C = Wc.shape[1]
    G = 128
    E = edge_index.shape[1]
    src = edge_index[0]
    dst = edge_index[1]
    NP = ((N + 127) // 128) * 128

    xt = _make_xt(N, NP, D)(x).reshape(-1)
    aggt1, deg = _make_agg(N, NP, D, E, True)(xt, src, dst)
    h1, h1t = _make_layer(N, NP, D, H)(
        aggt1.reshape(D, NP), deg.reshape(1, NP), x, Wl1, Wr1,
        bl1.reshape(1, H))
    aggt2 = _make_agg(N, NP, H, E, False)(h1t.reshape(-1), src, dst)
    out = _make_final(N, NP, H, G, C)(
        aggt2.reshape(H, NP), deg.reshape(1, NP), h1, Wl2, Wr2,
        bl2.reshape(1, H), batch.reshape(N, 1), Wc, bc.reshape(1, C))
    return out
